# baseline (speedup 1.0000x reference)
import functools

import jax
import jax.numpy as jnp
from jax import lax
from jax.experimental import pallas as pl
from jax.experimental.pallas import tpu as pltpu
from jax.experimental.pallas import tpu_sc as plsc

N = 10000
D = 128
E = 320000

NC = 2
NS = 16
NW = NC * NS
EPW = E // NW
KA = 40
NCA = EPW // KA
NBUF = 4
RCH = 400
NDC = N // RCH
RTC = 2000
RROW = KA
NRC = N // RROW

_mesh = plsc.VectorSubcoreMesh(core_axis_name="c", subcore_axis_name="s")


@functools.partial(
    pl.kernel,
    out_type=jax.ShapeDtypeStruct((NC * 3 * N,), jnp.float32),
    mesh=_mesh,
    scratch_types=[
        pltpu.VMEM_SHARED((N,), jnp.float32),
        pltpu.VMEM_SHARED((N,), jnp.float32),
        pltpu.VMEM_SHARED((N,), jnp.float32),
        pltpu.VMEM((EPW,), jnp.int32),
        pltpu.VMEM((EPW,), jnp.int32),
        pltpu.VMEM((EPW,), jnp.int32),
        pltpu.VMEM((EPW,), jnp.float32),
        pltpu.VMEM((RCH,), jnp.float32),
        pltpu.SemaphoreType.DMA,
        pltpu.SemaphoreType.DMA,
        pltpu.SemaphoreType.DMA,
    ],
)
def _deg_kernel(e0, e1, e2, out, d0_sh, d1_sh, d2_sh,
                idx0, idx1, idx2, onesv, zv, sem0, sem1, sem2):
    cid = lax.axis_index("c")
    sid = lax.axis_index("s")
    deg_sh = (d0_sh, d1_sh, d2_sh)
    idxb = (idx0, idx1, idx2)
    sems = (sem0, sem1, sem2)

    def ones_body(k, _):
        onesv[pl.ds(k * 16, 16)] = jnp.ones((16,), jnp.float32)
        return 0

    lax.fori_loop(0, EPW // 16, ones_body, 0)
    for k in range(RCH // 16):
        zv[pl.ds(k * 16, 16)] = jnp.zeros((16,), jnp.float32)

    for r in range(3):
        for k in range((NDC + NS - 1) // NS):
            ch = sid + NS * k

            @pl.when(ch < NDC)
            def _():
                pltpu.sync_copy(zv, deg_sh[r].at[pl.ds(ch * RCH, RCH)])
    plsc.subcore_barrier()

    base = (sid * NC + cid) * EPW
    for r, ei in enumerate((e0, e1, e2)):
        pltpu.async_copy(ei.at[pl.ds(base, EPW)], idxb[r], sems[r])
    for r, ei in enumerate((e0, e1, e2)):
        pltpu.make_async_copy(ei.at[pl.ds(base, EPW)], idxb[r], sems[r]).wait()
        pltpu.sync_copy(onesv, deg_sh[r].at[idxb[r]], add=True)
    plsc.subcore_barrier()

    for r in range(3):
        for k in range((NDC + NS - 1) // NS):
            ch = sid + NS * k

            @pl.when(ch < NDC)
            def _():
                pltpu.sync_copy(deg_sh[r].at[pl.ds(ch * RCH, RCH)], zv)
                pltpu.sync_copy(
                    zv, out.at[pl.ds((cid * 3 + r) * N + ch * RCH, RCH)]
                )


def _isq_block(degp_ref):
    deg = degp_ref[0, 0] + degp_ref[0, 1]
    return jnp.clip(deg, 1e-12, None) ** -0.5


def _tc_pre_body(degp_ref, x_ref, w0_ref, w1_ref, w2_ref,
                 z0_ref, z1_ref, z2_ref):
    x = x_ref[...]
    isq = _isq_block(degp_ref)
    for r, (z_ref, w_ref) in enumerate(
        ((z0_ref, w0_ref), (z1_ref, w1_ref), (z2_ref, w2_ref))
    ):
        z_ref[...] = jnp.dot(
            x * isq[r][:, None], w_ref[...],
            preferred_element_type=jnp.float32,
        )


def _tc_pre(degp4, X, W0, W1, W2):
    R = RTC
    f32 = jnp.float32
    return pl.pallas_call(
        _tc_pre_body,
        grid=(N // R,),
        in_specs=[
            pl.BlockSpec((1, NC, 3, R), lambda b: (b, 0, 0, 0)),
            pl.BlockSpec((R, D), lambda b: (b, 0)),
            pl.BlockSpec((D, D), lambda b: (0, 0)),
            pl.BlockSpec((D, D), lambda b: (0, 0)),
            pl.BlockSpec((D, D), lambda b: (0, 0)),
        ],
        out_specs=[
            pl.BlockSpec((R, D), lambda b: (b, 0)),
            pl.BlockSpec((R, D), lambda b: (b, 0)),
            pl.BlockSpec((R, D), lambda b: (b, 0)),
        ],
        out_shape=[
            jax.ShapeDtypeStruct((N, D), f32),
            jax.ShapeDtypeStruct((N, D), f32),
            jax.ShapeDtypeStruct((N, D), f32),
        ],
    )(degp4, X, W0, W1, W2)


@functools.partial(
    pl.kernel,
    out_type=jax.ShapeDtypeStruct((3, NC, N, D), jnp.float32),
    mesh=_mesh,
    scratch_types=[
        pltpu.VMEM_SHARED((N, D), jnp.float32),
        pltpu.VMEM((EPW,), jnp.int32),
        pltpu.VMEM((EPW,), jnp.int32),
        pltpu.VMEM((KA, D), jnp.float32),
        pltpu.VMEM((KA, D), jnp.float32),
        pltpu.VMEM((KA, D), jnp.float32),
        pltpu.VMEM((KA, D), jnp.float32),
        pltpu.SemaphoreType.DMA,
        pltpu.SemaphoreType.DMA,
        pltpu.SemaphoreType.DMA,
        pltpu.SemaphoreType.DMA,
        pltpu.SemaphoreType.DMA,
    ],
)
def _agg_kernel(z0, z1, z2, e0, e1, e2, out,
                acc_sh, iiv, jjv, rows0, rows1, rows2, rows3,
                semi, sem0, sem1, sem2, sem3):
    cid = lax.axis_index("c")
    sid = lax.axis_index("s")
    wid = sid * NC + cid
    base = wid * EPW
    rowsb = (rows0, rows1, rows2, rows3)
    sems = (sem0, sem1, sem2, sem3)

    for r, (z, ei) in enumerate(((z0, e0), (z1, e1), (z2, e2))):
        pltpu.async_copy(ei.at[pl.ds(base, EPW)], iiv, semi)
        pltpu.async_copy(ei.at[pl.ds(E + base, EPW)], jjv, semi)

        def zb(q, _):
            rows0[q // (D // 16), pl.ds((q % (D // 16)) * 16, 16)] = (
                jnp.zeros((16,), jnp.float32)
            )
            return 0

        lax.fori_loop(0, RROW * (D // 16), zb, 0)
        for k in range((NRC + NS - 1) // NS):
            ch = sid + NS * k

            @pl.when(ch < NRC)
            def _():
                pltpu.sync_copy(rows0, acc_sh.at[pl.ds(ch * RROW, RROW)])
        plsc.subcore_barrier()

        pltpu.make_async_copy(ei.at[pl.ds(base, EPW)], iiv, semi).wait()
        pltpu.make_async_copy(ei.at[pl.ds(base, EPW)], jjv, semi).wait()

        for b in range(NBUF):
            pltpu.async_copy(z.at[jjv.at[pl.ds(b * KA, KA)]],
                             rowsb[b], sems[b])

        def body(it, _, z=z):
            for b in range(NBUF):
                c = it * NBUF + b
                pltpu.make_async_copy(z.at[jjv.at[pl.ds(c * KA, KA)]],
                                      rowsb[b], sems[b]).wait()
                pltpu.sync_copy(rowsb[b],
                                acc_sh.at[iiv.at[pl.ds(c * KA, KA)]],
                                add=True)
                nxt = c + NBUF

                @pl.when(nxt < NCA)
                def _():
                    pltpu.async_copy(z.at[jjv.at[pl.ds(nxt * KA, KA)]],
                                     rowsb[b], sems[b])
            return 0

        lax.fori_loop(0, NCA // NBUF, body, 0)
        for tc in range((NCA // NBUF) * NBUF, NCA):
            b = tc % NBUF
            pltpu.make_async_copy(z.at[jjv.at[pl.ds(tc * KA, KA)]],
                                  rowsb[b], sems[b]).wait()
            pltpu.sync_copy(rowsb[b],
                            acc_sh.at[iiv.at[pl.ds(tc * KA, KA)]], add=True)
        plsc.subcore_barrier()

        for k in range((NRC + NS - 1) // NS):
            ch = sid + NS * k

            @pl.when(ch < NRC)
            def _():
                sl = pl.ds(ch * RROW, RROW)
                pltpu.sync_copy(acc_sh.at[sl], rows0)
                pltpu.sync_copy(rows0, out.at[r, cid, sl])
        plsc.subcore_barrier()


def _tc_post_body(degp_ref, p_ref, x_ref, ws_ref, b_ref, out_ref):
    isq = _isq_block(degp_ref)
    acc = (
        jnp.dot(x_ref[...], ws_ref[...], preferred_element_type=jnp.float32)
        + b_ref[...]
    )
    for r in range(3):
        acc = acc + isq[r][:, None] * (p_ref[r, 0] + p_ref[r, 1])
    out_ref[...] = acc


def _tc_post(degp4, partials, X, W_self, bias2):
    R = RTC
    return pl.pallas_call(
        _tc_post_body,
        grid=(N // R,),
        in_specs=[
            pl.BlockSpec((1, NC, 3, R), lambda b: (b, 0, 0, 0)),
            pl.BlockSpec((3, NC, R, D), lambda b: (0, 0, b, 0)),
            pl.BlockSpec((R, D), lambda b: (b, 0)),
            pl.BlockSpec((D, D), lambda b: (0, 0)),
            pl.BlockSpec((1, D), lambda b: (0, 0)),
        ],
        out_specs=pl.BlockSpec((R, D), lambda b: (b, 0)),
        out_shape=jax.ShapeDtypeStruct((N, D), jnp.float32),
    )(degp4, partials, X, W_self, bias2)


def kernel(X, edge_index_0, edge_index_1, edge_index_2,
           W_self, W0, W1, W2, bias):
    e0 = edge_index_0.reshape(2 * E)
    e1 = edge_index_1.reshape(2 * E)
    e2 = edge_index_2.reshape(2 * E)
    degp4 = (
        _deg_kernel(e0, e1, e2)
        .reshape(NC, 3, N // RTC, RTC)
        .transpose(2, 0, 1, 3)
    )
    z0, z1, z2 = _tc_pre(degp4, X, W0, W1, W2)
    partials = _agg_kernel(z0, z1, z2, e0, e1, e2)
    return _tc_post(degp4, partials, X, W_self, bias.reshape(1, D))


# speedup vs baseline: 1.0518x; 1.0518x over previous
import functools

import jax
import jax.numpy as jnp
from jax import lax
from jax.experimental import pallas as pl
from jax.experimental.pallas import tpu as pltpu
from jax.experimental.pallas import tpu_sc as plsc

N = 10000
D = 128
E = 320000

NC = 2
NS = 16
NW = NC * NS
EPW = E // NW
KA = 80
NCA = EPW // KA
NBUF = 3
RCH = 400
NDC = N // RCH
RTC = 2000
RROW = 80
NRC = N // RROW

_mesh = plsc.VectorSubcoreMesh(core_axis_name="c", subcore_axis_name="s")


@functools.partial(
    pl.kernel,
    out_type=jax.ShapeDtypeStruct((NC * 3 * N,), jnp.float32),
    mesh=_mesh,
    scratch_types=[
        pltpu.VMEM_SHARED((N,), jnp.float32),
        pltpu.VMEM_SHARED((N,), jnp.float32),
        pltpu.VMEM_SHARED((N,), jnp.float32),
        pltpu.VMEM((EPW,), jnp.int32),
        pltpu.VMEM((EPW,), jnp.int32),
        pltpu.VMEM((EPW,), jnp.int32),
        pltpu.VMEM((EPW,), jnp.float32),
        pltpu.VMEM((RCH,), jnp.float32),
        pltpu.SemaphoreType.DMA,
        pltpu.SemaphoreType.DMA,
        pltpu.SemaphoreType.DMA,
    ],
)
def _deg_kernel(e0, e1, e2, out, d0_sh, d1_sh, d2_sh,
                idx0, idx1, idx2, onesv, zv, sem0, sem1, sem2):
    cid = lax.axis_index("c")
    sid = lax.axis_index("s")
    deg_sh = (d0_sh, d1_sh, d2_sh)
    idxb = (idx0, idx1, idx2)
    sems = (sem0, sem1, sem2)

    def ones_body(k, _):
        onesv[pl.ds(k * 16, 16)] = jnp.ones((16,), jnp.float32)
        return 0

    lax.fori_loop(0, EPW // 16, ones_body, 0)
    for k in range(RCH // 16):
        zv[pl.ds(k * 16, 16)] = jnp.zeros((16,), jnp.float32)

    for r in range(3):
        for k in range((NDC + NS - 1) // NS):
            ch = sid + NS * k

            @pl.when(ch < NDC)
            def _():
                pltpu.sync_copy(zv, deg_sh[r].at[pl.ds(ch * RCH, RCH)])
    plsc.subcore_barrier()

    base = (sid * NC + cid) * EPW
    for r, ei in enumerate((e0, e1, e2)):
        pltpu.async_copy(ei.at[pl.ds(base, EPW)], idxb[r], sems[r])
    for r, ei in enumerate((e0, e1, e2)):
        pltpu.make_async_copy(ei.at[pl.ds(base, EPW)], idxb[r], sems[r]).wait()
        pltpu.sync_copy(onesv, deg_sh[r].at[idxb[r]], add=True)
    plsc.subcore_barrier()

    for r in range(3):
        for k in range((NDC + NS - 1) // NS):
            ch = sid + NS * k

            @pl.when(ch < NDC)
            def _():
                pltpu.sync_copy(deg_sh[r].at[pl.ds(ch * RCH, RCH)], zv)
                pltpu.sync_copy(
                    zv, out.at[pl.ds((cid * 3 + r) * N + ch * RCH, RCH)]
                )


def _isq_block(degp_ref):
    deg = degp_ref[0, 0] + degp_ref[0, 1]
    return jnp.clip(deg, 1e-12, None) ** -0.5


def _tc_pre_body(degp_ref, x_ref, w0_ref, w1_ref, w2_ref,
                 z0_ref, z1_ref, z2_ref):
    x = x_ref[...]
    isq = _isq_block(degp_ref)
    for r, (z_ref, w_ref) in enumerate(
        ((z0_ref, w0_ref), (z1_ref, w1_ref), (z2_ref, w2_ref))
    ):
        z_ref[...] = jnp.dot(
            x * isq[r][:, None], w_ref[...],
            preferred_element_type=jnp.float32,
        )


def _tc_pre(degp4, X, W0, W1, W2):
    R = RTC
    f32 = jnp.float32
    return pl.pallas_call(
        _tc_pre_body,
        grid=(N // R,),
        in_specs=[
            pl.BlockSpec((1, NC, 3, R), lambda b: (b, 0, 0, 0)),
            pl.BlockSpec((R, D), lambda b: (b, 0)),
            pl.BlockSpec((D, D), lambda b: (0, 0)),
            pl.BlockSpec((D, D), lambda b: (0, 0)),
            pl.BlockSpec((D, D), lambda b: (0, 0)),
        ],
        out_specs=[
            pl.BlockSpec((R, D), lambda b: (b, 0)),
            pl.BlockSpec((R, D), lambda b: (b, 0)),
            pl.BlockSpec((R, D), lambda b: (b, 0)),
        ],
        out_shape=[
            jax.ShapeDtypeStruct((N, D), f32),
            jax.ShapeDtypeStruct((N, D), f32),
            jax.ShapeDtypeStruct((N, D), f32),
        ],
    )(degp4, X, W0, W1, W2)


@functools.partial(
    pl.kernel,
    out_type=jax.ShapeDtypeStruct((3, NC, N, D), jnp.float32),
    mesh=_mesh,
    scratch_types=[
        pltpu.VMEM_SHARED((N, D), jnp.float32),
        pltpu.VMEM((EPW,), jnp.int32),
        pltpu.VMEM((EPW,), jnp.int32),
        pltpu.VMEM((KA, D), jnp.float32),
        pltpu.VMEM((KA, D), jnp.float32),
        pltpu.VMEM((KA, D), jnp.float32),
        pltpu.SemaphoreType.DMA,
        pltpu.SemaphoreType.DMA,
        pltpu.SemaphoreType.DMA,
        pltpu.SemaphoreType.DMA,
    ],
)
def _agg_kernel(z0, z1, z2, e0, e1, e2, out,
                acc_sh, iiv, jjv, rows0, rows1, rows2,
                semi, sem0, sem1, sem2):
    cid = lax.axis_index("c")
    sid = lax.axis_index("s")
    wid = sid * NC + cid
    base = wid * EPW
    rowsb = (rows0, rows1, rows2)
    sems = (sem0, sem1, sem2)

    for r, (z, ei) in enumerate(((z0, e0), (z1, e1), (z2, e2))):
        pltpu.async_copy(ei.at[pl.ds(base, EPW)], iiv, semi)
        pltpu.async_copy(ei.at[pl.ds(E + base, EPW)], jjv, semi)

        def zb(q, _):
            rows0[q // (D // 16), pl.ds((q % (D // 16)) * 16, 16)] = (
                jnp.zeros((16,), jnp.float32)
            )
            return 0

        lax.fori_loop(0, RROW * (D // 16), zb, 0)
        for k in range((NRC + NS - 1) // NS):
            ch = sid + NS * k

            @pl.when(ch < NRC)
            def _():
                pltpu.sync_copy(rows0, acc_sh.at[pl.ds(ch * RROW, RROW)])
        plsc.subcore_barrier()

        pltpu.make_async_copy(ei.at[pl.ds(base, EPW)], iiv, semi).wait()
        pltpu.make_async_copy(ei.at[pl.ds(base, EPW)], jjv, semi).wait()

        for b in range(NBUF):
            pltpu.async_copy(z.at[jjv.at[pl.ds(b * KA, KA)]],
                             rowsb[b], sems[b])

        def body(it, _, z=z):
            for b in range(NBUF):
                c = it * NBUF + b
                pltpu.make_async_copy(z.at[jjv.at[pl.ds(c * KA, KA)]],
                                      rowsb[b], sems[b]).wait()
                pltpu.sync_copy(rowsb[b],
                                acc_sh.at[iiv.at[pl.ds(c * KA, KA)]],
                                add=True)
                nxt = c + NBUF

                @pl.when(nxt < NCA)
                def _():
                    pltpu.async_copy(z.at[jjv.at[pl.ds(nxt * KA, KA)]],
                                     rowsb[b], sems[b])
            return 0

        lax.fori_loop(0, NCA // NBUF, body, 0)
        for tc in range((NCA // NBUF) * NBUF, NCA):
            b = tc % NBUF
            pltpu.make_async_copy(z.at[jjv.at[pl.ds(tc * KA, KA)]],
                                  rowsb[b], sems[b]).wait()
            pltpu.sync_copy(rowsb[b],
                            acc_sh.at[iiv.at[pl.ds(tc * KA, KA)]], add=True)
        plsc.subcore_barrier()

        for k in range((NRC + NS - 1) // NS):
            ch = sid + NS * k
            b = k % 2

            @pl.when(ch < NRC)
            def _():
                sl = pl.ds(ch * RROW, RROW)
                if k >= 2:
                    pltpu.make_async_copy(
                        rowsb[b], out.at[r, cid, sl], sems[b]
                    ).wait()
                pltpu.sync_copy(acc_sh.at[sl], rowsb[b])
                pltpu.async_copy(rowsb[b], out.at[r, cid, sl], sems[b])
        for b in range(2):
            pltpu.make_async_copy(
                rowsb[b], out.at[r, cid, pl.ds(0, RROW)], sems[b]
            ).wait()
        plsc.subcore_barrier()


def _tc_post_body(degp_ref, p_ref, x_ref, ws_ref, b_ref, out_ref):
    isq = _isq_block(degp_ref)
    acc = (
        jnp.dot(x_ref[...], ws_ref[...], preferred_element_type=jnp.float32)
        + b_ref[...]
    )
    for r in range(3):
        acc = acc + isq[r][:, None] * (p_ref[r, 0] + p_ref[r, 1])
    out_ref[...] = acc


def _tc_post(degp4, partials, X, W_self, bias2):
    R = RTC
    return pl.pallas_call(
        _tc_post_body,
        grid=(N // R,),
        in_specs=[
            pl.BlockSpec((1, NC, 3, R), lambda b: (b, 0, 0, 0)),
            pl.BlockSpec((3, NC, R, D), lambda b: (0, 0, b, 0)),
            pl.BlockSpec((R, D), lambda b: (b, 0)),
            pl.BlockSpec((D, D), lambda b: (0, 0)),
            pl.BlockSpec((1, D), lambda b: (0, 0)),
        ],
        out_specs=pl.BlockSpec((R, D), lambda b: (b, 0)),
        out_shape=jax.ShapeDtypeStruct((N, D), jnp.float32),
    )(degp4, partials, X, W_self, bias2)


def kernel(X, edge_index_0, edge_index_1, edge_index_2,
           W_self, W0, W1, W2, bias):
    e0 = edge_index_0.reshape(2 * E)
    e1 = edge_index_1.reshape(2 * E)
    e2 = edge_index_2.reshape(2 * E)
    degp4 = (
        _deg_kernel(e0, e1, e2)
        .reshape(NC, 3, N // RTC, RTC)
        .transpose(2, 0, 1, 3)
    )
    z0, z1, z2 = _tc_pre(degp4, X, W0, W1, W2)
    partials = _agg_kernel(z0, z1, z2, e0, e1, e2)
    return _tc_post(degp4, partials, X, W_self, bias.reshape(1, D))
